# baseline (speedup 1.0000x reference)
import functools
import math

import jax
import jax.numpy as jnp
from jax import lax
from jax.experimental import pallas as pl
from jax.experimental.pallas import tpu as pltpu
from jax.experimental.pallas import tpu_sc as plsc

D_MODEL = 768
SCALE = math.sqrt(float(D_MODEL))

_NG = 2
_NS = 2
_CHUNK = 16
_LCM = 2


@functools.cache
def _build(BATCH: int, SEQ: int, V: int, D: int):
    info = plsc.get_sparse_core_info()
    NC, NS_sub, L = info.num_cores, info.num_subcores, info.num_lanes
    NW = NC * NS_sub
    B = BATCH * SEQ
    assert B % NW == 0 and SEQ % (B // NW) == 0
    b_per_w = B // NW
    w_per_s = SEQ // b_per_w
    C = _CHUNK
    assert b_per_w % C == 0 and D % L == 0
    n_chunks = b_per_w // C
    n_slices = D // L

    mesh = plsc.VectorSubcoreMesh(core_axis_name="c", subcore_axis_name="s")

    @functools.partial(
        pl.kernel,
        mesh=mesh,
        out_type=jax.ShapeDtypeStruct((BATCH, SEQ, D), jnp.float32),
        scratch_types=[
            pltpu.VMEM((b_per_w,), jnp.int32),
            [pltpu.VMEM((C, D), jnp.float32) for _ in range(_NG)],
            [pltpu.VMEM((C, D), jnp.float32) for _ in range(_NS)],
            [pltpu.SemaphoreType.DMA for _ in range(_NG)],
            [pltpu.SemaphoreType.DMA for _ in range(_NS)],
        ],
    )
    def emb_kernel(x_hbm, table_hbm, out_hbm, idx_v, ibufs, obufs, gsems, ssems):
        wid = lax.axis_index("s") * NC + lax.axis_index("c")
        bi = wid // w_per_s
        r0 = (wid % w_per_s) * b_per_w

        pltpu.sync_copy(x_hbm.at[bi, pl.ds(r0, b_per_w)], idx_v)

        def gather(g, bg):
            return pltpu.make_async_copy(
                table_hbm.at[idx_v.at[pl.ds(g * C, C)]], ibufs[bg], gsems[bg]
            )

        def scatter(g, bo):
            return pltpu.make_async_copy(
                obufs[bo], out_hbm.at[bi, pl.ds(r0 + g * C, C)], ssems[bo]
            )

        def scale_chunk(bg, bo):
            def row(r, carry):
                for j in range(n_slices):
                    sl = (r, pl.ds(j * L, L))
                    obufs[bo][sl] = ibufs[bg][sl] * SCALE
                return carry

            lax.fori_loop(0, C, row, 0, unroll=False)

        def process(g, bg, bo, wait_s, next_g):
            gather(g, bg).wait()
            if wait_s:
                scatter(g - _NS, bo).wait()
            scale_chunk(bg, bo)
            scatter(g, bo).start()
            if next_g:
                gather(g + _NG, bg).start()

        for g in range(_NG):
            gather(g, g % _NG).start()

        for g in range(_NG):
            process(g, g % _NG, g % _NS, wait_s=(g >= _NS), next_g=True)

        n_steady = (n_chunks - _NG - (_LCM - 1)) // _LCM
        steady_end = _NG + n_steady * _LCM

        def outer(k, carry):
            g0 = _NG + k * _LCM
            for j in range(_LCM):
                g = g0 + j
                process(g, (_NG + j) % _NG, (_NG + j) % _NS, wait_s=True,
                        next_g=True)
            return carry

        lax.fori_loop(0, n_steady, outer, 0, unroll=False)

        for g in range(steady_end, n_chunks):
            process(g, g % _NG, g % _NS, wait_s=True,
                    next_g=(g + _NG < n_chunks))

        for g in range(n_chunks - _NS, n_chunks):
            scatter(g, g % _NS).wait()

    return emb_kernel


def kernel(x, table):
    BATCH, SEQ = x.shape
    V, D = table.shape
    return _build(BATCH, SEQ, V, D)(x.astype(jnp.int32), table)


# speedup vs baseline: 1.0490x; 1.0490x over previous
import functools
import math

import jax
import jax.numpy as jnp
from jax import lax
from jax.experimental import pallas as pl
from jax.experimental.pallas import tpu as pltpu
from jax.experimental.pallas import tpu_sc as plsc

D_MODEL = 768
SCALE = math.sqrt(float(D_MODEL))

_NG = 2
_NS = 2
_CHUNK = 32
_LCM = 2


@functools.cache
def _build(BATCH: int, SEQ: int, V: int, D: int):
    info = plsc.get_sparse_core_info()
    NC, NS_sub, L = info.num_cores, info.num_subcores, info.num_lanes
    NW = NC * NS_sub
    B = BATCH * SEQ
    assert B % NW == 0 and SEQ % (B // NW) == 0
    b_per_w = B // NW
    w_per_s = SEQ // b_per_w
    C = _CHUNK
    assert b_per_w % C == 0 and D % L == 0
    n_chunks = b_per_w // C
    n_slices = D // L

    mesh = plsc.VectorSubcoreMesh(core_axis_name="c", subcore_axis_name="s")

    @functools.partial(
        pl.kernel,
        mesh=mesh,
        out_type=jax.ShapeDtypeStruct((BATCH, SEQ, D), jnp.float32),
        scratch_types=[
            pltpu.VMEM((b_per_w,), jnp.int32),
            [pltpu.VMEM((C, D), jnp.float32) for _ in range(_NG)],
            [pltpu.VMEM((C, D), jnp.float32) for _ in range(_NS)],
            [pltpu.SemaphoreType.DMA for _ in range(_NG)],
            [pltpu.SemaphoreType.DMA for _ in range(_NS)],
        ],
    )
    def emb_kernel(x_hbm, table_hbm, out_hbm, idx_v, ibufs, obufs, gsems, ssems):
        wid = lax.axis_index("s") * NC + lax.axis_index("c")
        bi = wid // w_per_s
        r0 = (wid % w_per_s) * b_per_w

        pltpu.sync_copy(x_hbm.at[bi, pl.ds(r0, b_per_w)], idx_v)

        def gather(g, bg):
            return pltpu.make_async_copy(
                table_hbm.at[idx_v.at[pl.ds(g * C, C)]], ibufs[bg], gsems[bg]
            )

        def scatter(g, bo):
            return pltpu.make_async_copy(
                obufs[bo], out_hbm.at[bi, pl.ds(r0 + g * C, C)], ssems[bo]
            )

        def scale_chunk(bg, bo):
            def row(r, carry):
                for j in range(n_slices):
                    sl = (r, pl.ds(j * L, L))
                    obufs[bo][sl] = ibufs[bg][sl] * SCALE
                return carry

            lax.fori_loop(0, C, row, 0, unroll=False)

        def process(g, bg, bo, wait_s, next_g):
            gather(g, bg).wait()
            if wait_s:
                scatter(g - _NS, bo).wait()
            scale_chunk(bg, bo)
            scatter(g, bo).start()
            if next_g:
                gather(g + _NG, bg).start()

        for g in range(_NG):
            gather(g, g % _NG).start()

        for g in range(_NG):
            process(g, g % _NG, g % _NS, wait_s=(g >= _NS), next_g=True)

        n_steady = (n_chunks - _NG - (_LCM - 1)) // _LCM
        steady_end = _NG + n_steady * _LCM

        def outer(k, carry):
            g0 = _NG + k * _LCM
            for j in range(_LCM):
                g = g0 + j
                process(g, (_NG + j) % _NG, (_NG + j) % _NS, wait_s=True,
                        next_g=True)
            return carry

        lax.fori_loop(0, n_steady, outer, 0, unroll=False)

        for g in range(steady_end, n_chunks):
            process(g, g % _NG, g % _NS, wait_s=True,
                    next_g=(g + _NG < n_chunks))

        for g in range(n_chunks - _NS, n_chunks):
            scatter(g, g % _NS).wait()

    return emb_kernel


def kernel(x, table):
    BATCH, SEQ = x.shape
    V, D = table.shape
    return _build(BATCH, SEQ, V, D)(x.astype(jnp.int32), table)
